# baseline (speedup 1.0000x reference)
import functools

import jax
import jax.numpy as jnp
from jax.experimental import pallas as pl
from jax.experimental.pallas import tpu as pltpu

_E = 8
_NEG_INF = -1e30


def _moe_body(x_ref, Wg_ref, bg_ref, We_ref, be_ref, out_ref, aux_ref,
              w_scr, xbf_scr, probs_acc, cnt_acc, *, blk, n_tokens):
    t = pl.program_id(0)
    e = pl.program_id(1)
    nt = pl.num_programs(0)

    @pl.when(e == 0)
    def _gate():
        xb = x_ref[...]
        xbf_scr[...] = xb.astype(jnp.bfloat16)
        logits = jax.lax.dot_general(
            xb, Wg_ref[...], (((1,), (0,)), ((), ())),
            preferred_element_type=jnp.float32) + bg_ref[...]
        iota_e = jax.lax.broadcasted_iota(jnp.int32, (blk, _E), 1)
        max1 = jnp.max(logits, axis=1, keepdims=True)
        idx1 = jnp.min(jnp.where(logits == max1, iota_e, _E), axis=1,
                       keepdims=True)
        masked = jnp.where(iota_e == idx1, _NEG_INF, logits)
        max2 = jnp.max(masked, axis=1, keepdims=True)
        idx2 = jnp.min(jnp.where(masked == max2, iota_e, _E), axis=1,
                       keepdims=True)
        e2 = jnp.exp(max2 - max1)
        w1 = 1.0 / (1.0 + e2)
        w2 = 1.0 - w1
        w_scr[...] = (jnp.where(iota_e == idx1, w1, 0.0) +
                      jnp.where(iota_e == idx2, w2, 0.0))
        probs = jnp.exp(logits - max1)
        probs = probs / jnp.sum(probs, axis=1, keepdims=True)
        block_probs = jnp.sum(probs, axis=0, keepdims=True)
        block_cnt = jnp.sum((iota_e == idx1).astype(jnp.float32), axis=0,
                            keepdims=True)

        @pl.when(t == 0)
        def _init():
            probs_acc[...] = block_probs
            cnt_acc[...] = block_cnt

        @pl.when(t > 0)
        def _accum():
            probs_acc[...] += block_probs
            cnt_acc[...] += block_cnt

    acc = jax.lax.dot_general(
        xbf_scr[...], We_ref[0], (((1,), (0,)), ((), ())),
        preferred_element_type=jnp.float32)
    onehot = (jax.lax.broadcasted_iota(jnp.int32, (_E, 1), 0) == e
              ).astype(jnp.float32)
    w_col = jax.lax.dot_general(w_scr[...], onehot, (((1,), (0,)), ((), ())),
                                preferred_element_type=jnp.float32)
    contrib = (acc + be_ref[0]) * w_col

    @pl.when(e == 0)
    def _first():
        out_ref[...] = contrib

    @pl.when(e > 0)
    def _rest():
        out_ref[...] += contrib

    @pl.when((t == nt - 1) & (e == _E - 1))
    def _aux():
        tokens_per_expert = cnt_acc[...]
        avg_prob = probs_acc[...] / n_tokens
        aux_ref[...] = jnp.sum(
            tokens_per_expert / (n_tokens + 1e-8) * avg_prob,
            axis=1, keepdims=True) * _E


def kernel(x, Wg, bg, We, be):
    n, d = x.shape
    blk = 2048
    nt = n // blk
    grid = (nt, _E)
    body = functools.partial(_moe_body, blk=blk, n_tokens=n)
    out, aux = pl.pallas_call(
        body,
        grid=grid,
        in_specs=[
            pl.BlockSpec((blk, d), lambda t, e: (t, 0)),
            pl.BlockSpec((d, _E), lambda t, e: (0, 0)),
            pl.BlockSpec((1, _E), lambda t, e: (0, 0)),
            pl.BlockSpec((1, d, d), lambda t, e: (e, 0, 0)),
            pl.BlockSpec((1, 1, d), lambda t, e: (e, 0, 0)),
        ],
        out_specs=[
            pl.BlockSpec((blk, d), lambda t, e: (t, 0)),
            pl.BlockSpec((1, 1), lambda t, e: (0, 0)),
        ],
        out_shape=[
            jax.ShapeDtypeStruct((n, d), jnp.float32),
            jax.ShapeDtypeStruct((1, 1), jnp.float32),
        ],
        scratch_shapes=[
            pltpu.VMEM((blk, _E), jnp.float32),
            pltpu.VMEM((blk, d), jnp.bfloat16),
            pltpu.VMEM((1, _E), jnp.float32),
            pltpu.VMEM((1, _E), jnp.float32),
        ],
        compiler_params=pltpu.CompilerParams(
            dimension_semantics=("arbitrary", "arbitrary")),
    )(x, Wg, bg.reshape(1, _E), We.astype(jnp.bfloat16),
      be.reshape(_E, 1, d))
    return out, aux[0, 0]


# speedup vs baseline: 1.1936x; 1.1936x over previous
import functools

import jax
import jax.numpy as jnp
from jax.experimental import pallas as pl
from jax.experimental.pallas import tpu as pltpu

_E = 8
_NEG_INF = -1e30


def _moe_body(x_ref, Wg_ref, bg_ref, WeK_ref, be_ref, out_ref, aux_ref,
              probs_acc, cnt_acc, *, blk, n_tokens):
    t = pl.program_id(0)
    nt = pl.num_programs(0)

    xb = x_ref[...]
    logits = jax.lax.dot_general(
        xb, Wg_ref[...], (((1,), (0,)), ((), ())),
        preferred_element_type=jnp.float32) + bg_ref[...]
    iota_e = jax.lax.broadcasted_iota(jnp.int32, (blk, _E), 1)
    max1 = jnp.max(logits, axis=1, keepdims=True)
    idx1 = jnp.min(jnp.where(logits == max1, iota_e, _E), axis=1,
                   keepdims=True)
    masked = jnp.where(iota_e == idx1, _NEG_INF, logits)
    max2 = jnp.max(masked, axis=1, keepdims=True)
    idx2 = jnp.min(jnp.where(masked == max2, iota_e, _E), axis=1,
                   keepdims=True)
    e2 = jnp.exp(max2 - max1)
    w1 = 1.0 / (1.0 + e2)
    w2 = 1.0 - w1
    w_dense = (jnp.where(iota_e == idx1, w1, 0.0) +
               jnp.where(iota_e == idx2, w2, 0.0))

    probs = jnp.exp(logits - max1)
    probs = probs / jnp.sum(probs, axis=1, keepdims=True)
    block_probs = jnp.sum(probs, axis=0, keepdims=True)
    block_cnt = jnp.sum((iota_e == idx1).astype(jnp.float32), axis=0,
                        keepdims=True)

    @pl.when(t == 0)
    def _init():
        probs_acc[...] = block_probs
        cnt_acc[...] = block_cnt

    @pl.when(t > 0)
    def _accum():
        probs_acc[...] += block_probs
        cnt_acc[...] += block_cnt

    xw = jnp.concatenate(
        [(xb * w_dense[:, e:e + 1]).astype(jnp.bfloat16) for e in range(_E)],
        axis=1)
    acc = jax.lax.dot_general(
        xw, WeK_ref[...], (((1,), (0,)), ((), ())),
        preferred_element_type=jnp.float32)
    bias = jax.lax.dot_general(
        w_dense, be_ref[...], (((1,), (0,)), ((), ())),
        preferred_element_type=jnp.float32)
    out_ref[...] = acc + bias

    @pl.when(t == nt - 1)
    def _aux():
        aux_ref[...] = jnp.sum(
            cnt_acc[...] / (n_tokens + 1e-8) * (probs_acc[...] / n_tokens),
            axis=1, keepdims=True) * _E


def kernel(x, Wg, bg, We, be):
    n, d = x.shape
    blk = 1024
    nt = n // blk
    body = functools.partial(_moe_body, blk=blk, n_tokens=n)
    out, aux = pl.pallas_call(
        body,
        grid=(nt,),
        in_specs=[
            pl.BlockSpec((blk, d), lambda t: (t, 0)),
            pl.BlockSpec((d, _E), lambda t: (0, 0)),
            pl.BlockSpec((1, _E), lambda t: (0, 0)),
            pl.BlockSpec((_E * d, d), lambda t: (0, 0)),
            pl.BlockSpec((_E, d), lambda t: (0, 0)),
        ],
        out_specs=[
            pl.BlockSpec((blk, d), lambda t: (t, 0)),
            pl.BlockSpec((1, 1), lambda t: (0, 0)),
        ],
        out_shape=[
            jax.ShapeDtypeStruct((n, d), jnp.float32),
            jax.ShapeDtypeStruct((1, 1), jnp.float32),
        ],
        scratch_shapes=[
            pltpu.VMEM((1, _E), jnp.float32),
            pltpu.VMEM((1, _E), jnp.float32),
        ],
        compiler_params=pltpu.CompilerParams(
            dimension_semantics=("arbitrary",)),
    )(x, Wg, bg.reshape(1, _E), We.reshape(_E * d, d).astype(jnp.bfloat16),
      be)
    return out, aux[0, 0]


# speedup vs baseline: 1.2082x; 1.0122x over previous
import functools

import jax
import jax.numpy as jnp
from jax.experimental import pallas as pl
from jax.experimental.pallas import tpu as pltpu

_E = 8
_NEG_INF = -1e30


def _gate_body(x_ref, Wg_ref, bg_ref, w_ref, aux_ref, probs_acc, cnt_acc,
               *, blk, n_tokens):
    t = pl.program_id(0)
    nt = pl.num_programs(0)

    logits = jax.lax.dot_general(
        x_ref[...], Wg_ref[...], (((1,), (0,)), ((), ())),
        preferred_element_type=jnp.float32) + bg_ref[...]
    iota_e = jax.lax.broadcasted_iota(jnp.int32, (blk, _E), 1)
    max1 = jnp.max(logits, axis=1, keepdims=True)
    idx1 = jnp.min(jnp.where(logits == max1, iota_e, _E), axis=1,
                   keepdims=True)
    masked = jnp.where(iota_e == idx1, _NEG_INF, logits)
    max2 = jnp.max(masked, axis=1, keepdims=True)
    idx2 = jnp.min(jnp.where(masked == max2, iota_e, _E), axis=1,
                   keepdims=True)
    e2 = jnp.exp(max2 - max1)
    w1 = 1.0 / (1.0 + e2)
    w2 = 1.0 - w1
    w_ref[...] = (jnp.where(iota_e == idx1, w1, 0.0) +
                  jnp.where(iota_e == idx2, w2, 0.0))

    probs = jnp.exp(logits - max1)
    probs = probs / jnp.sum(probs, axis=1, keepdims=True)
    block_probs = jnp.sum(probs, axis=0, keepdims=True)
    block_cnt = jnp.sum((iota_e == idx1).astype(jnp.float32), axis=0,
                        keepdims=True)

    @pl.when(t == 0)
    def _init():
        probs_acc[...] = block_probs
        cnt_acc[...] = block_cnt

    @pl.when(t > 0)
    def _accum():
        probs_acc[...] += block_probs
        cnt_acc[...] += block_cnt

    @pl.when(t == nt - 1)
    def _aux():
        aux_ref[...] = jnp.sum(
            cnt_acc[...] / (n_tokens + 1e-8) * (probs_acc[...] / n_tokens),
            axis=1, keepdims=True) * _E


def _expert_body(x_ref, w_ref, WeK_ref, be_ref, out_ref):
    xb = x_ref[...]
    w_dense = w_ref[...]
    xw = jnp.concatenate(
        [(xb * w_dense[:, e:e + 1]).astype(jnp.bfloat16) for e in range(_E)],
        axis=1)
    acc = jax.lax.dot_general(
        xw, WeK_ref[...], (((1,), (0,)), ((), ())),
        preferred_element_type=jnp.float32)
    bias = jax.lax.dot_general(
        w_dense, be_ref[...], (((1,), (0,)), ((), ())),
        preferred_element_type=jnp.float32)
    out_ref[...] = acc + bias


def kernel(x, Wg, bg, We, be):
    n, d = x.shape
    gblk = 4096
    gate_body = functools.partial(_gate_body, blk=gblk, n_tokens=n)
    w_dense, aux = pl.pallas_call(
        gate_body,
        grid=(n // gblk,),
        in_specs=[
            pl.BlockSpec((gblk, d), lambda t: (t, 0)),
            pl.BlockSpec((d, _E), lambda t: (0, 0)),
            pl.BlockSpec((1, _E), lambda t: (0, 0)),
        ],
        out_specs=[
            pl.BlockSpec((gblk, _E), lambda t: (t, 0)),
            pl.BlockSpec((1, 1), lambda t: (0, 0)),
        ],
        out_shape=[
            jax.ShapeDtypeStruct((n, _E), jnp.float32),
            jax.ShapeDtypeStruct((1, 1), jnp.float32),
        ],
        scratch_shapes=[
            pltpu.VMEM((1, _E), jnp.float32),
            pltpu.VMEM((1, _E), jnp.float32),
        ],
        compiler_params=pltpu.CompilerParams(
            dimension_semantics=("arbitrary",)),
    )(x, Wg, bg.reshape(1, _E))

    blk = 1024
    out = pl.pallas_call(
        _expert_body,
        grid=(n // blk,),
        in_specs=[
            pl.BlockSpec((blk, d), lambda t: (t, 0)),
            pl.BlockSpec((blk, _E), lambda t: (t, 0)),
            pl.BlockSpec((_E * d, d), lambda t: (0, 0)),
            pl.BlockSpec((_E, d), lambda t: (0, 0)),
        ],
        out_specs=pl.BlockSpec((blk, d), lambda t: (t, 0)),
        out_shape=jax.ShapeDtypeStruct((n, d), jnp.float32),
        compiler_params=pltpu.CompilerParams(
            dimension_semantics=("arbitrary",)),
    )(x, w_dense, We.reshape(_E * d, d).astype(jnp.bfloat16), be)
    return out, aux[0, 0]
